# baseline (speedup 1.0000x reference)
import functools

import jax
import jax.numpy as jnp
from jax import lax
from jax.experimental import pallas as pl
from jax.experimental.pallas import tpu as pltpu
from jax.experimental.pallas import tpu_sc as plsc

L = 16
KC = 384


def _build_transpose(V, D, NC, NS):
    NW = NC * NS
    nblk = V // KC
    VT = nblk * KC
    tail = V - VT
    trips = (nblk + NW - 1) // NW
    if trips % 2:
        trips += 1
    ntj = KC // 128
    ntr = D // 8

    mesh = plsc.VectorSubcoreMesh(core_axis_name="c", subcore_axis_name="s")

    @functools.partial(
        pl.kernel,
        mesh=mesh,
        out_type=jax.ShapeDtypeStruct((V * D,), jnp.float32),
        scratch_types=[
            pltpu.VMEM((ntr * ntj * 8, 128), jnp.float32),
            pltpu.VMEM((ntr * ntj * 8, 128), jnp.float32),
            pltpu.VMEM((KC * D,), jnp.float32),
            pltpu.VMEM((KC * D,), jnp.float32),
            pltpu.VMEM((max(tail, 1) * D,), jnp.float32),
            pltpu.SemaphoreType.DMA,
            pltpu.SemaphoreType.DMA,
        ],
        compiler_params=pltpu.CompilerParams(needs_layout_passes=False),
    )
    def transpose_kernel(tt_hbm, tail_hbm, out_hbm,
                         tb0, tb1, ob0, ob1, tlb, sem_i, sem_o):
        wid = lax.axis_index("s") * NC + lax.axis_index("c")
        tb = (tb0, tb1)
        ob = (ob0, ob1)
        iota = lax.iota(jnp.int32, L)
        rowv = [((2 * g + (iota >> 3)) * ntj) * 8 + (iota & 7)
                for g in range(D // L)]

        def blk(i):
            return wid + NW * i

        def tiles(i, buf):
            c0 = blk(i) * KC
            pairs = []
            for tr in range(ntr):
                for j in range(ntj):
                    pairs.append((tt_hbm.at[pl.ds(tr * 8, 8),
                                            pl.ds(c0 + j * 128, 128)],
                                  buf.at[pl.ds((tr * ntj + j) * 8, 8), :]))
            return pairs

        def start_in(i, buf):
            @pl.when(blk(i) < nblk)
            def _():
                for src, dst in tiles(i, buf):
                    pltpu.async_copy(src, dst, sem_i)

        def wait_in(i, buf):
            @pl.when(blk(i) < nblk)
            def _():
                for src, dst in tiles(i, buf):
                    pltpu.make_async_copy(src, dst, sem_i).wait()

        def start_out(i, buf):
            @pl.when(blk(i) < nblk)
            def _():
                pltpu.async_copy(
                    buf, out_hbm.at[pl.ds(blk(i) * KC * D, KC * D)], sem_o)

        def wait_out(i, buf):
            @pl.when(blk(i) < nblk)
            def _():
                pltpu.make_async_copy(
                    buf, out_hbm.at[pl.ds(blk(i) * KC * D, KC * D)],
                    sem_o).wait()

        def transpose_block(src, dst):
            def col(cq, c):
                jb = jnp.broadcast_to((cq >> 7) * 8, (L,))
                cb = jnp.broadcast_to(cq & 127, (L,))
                for g in range(D // L):
                    gv = plsc.load_gather(src, [rowv[g] + jb, cb])
                    dst[pl.ds(cq * D + g * L, L)] = gv
                return c
            lax.fori_loop(0, KC, col, 0)

        if tail:
            @pl.when(wid == NW - 1)
            def _():
                pltpu.sync_copy(tail_hbm, tlb)
                pltpu.sync_copy(tlb, out_hbm.at[pl.ds(VT * D, tail * D)])

        start_in(0, tb[0])

        def pair(t, carry):
            for k in range(2):
                i = 2 * t + k
                p = k
                start_in(i + 1, tb[1 - p])
                wait_in(i, tb[p])
                lax.cond(i >= 2, lambda: wait_out(i - 2, ob[p]),
                         lambda: None)
                transpose_block(tb[p], ob[p])
                start_out(i, ob[p])
            return carry

        lax.fori_loop(0, trips // 2, pair, 0)
        wait_out(trips - 2, ob[0])
        wait_out(trips - 1, ob[1])

    return transpose_kernel


def _build_gather(B, S, D, V, NC, NS):
    NW = NC * NS
    BPW = B // NW

    mesh = plsc.VectorSubcoreMesh(core_axis_name="c", subcore_axis_name="s")

    @functools.partial(
        pl.kernel,
        mesh=mesh,
        out_type=jax.ShapeDtypeStruct((S, D // 8, B // 128, 8, 128),
                                      jnp.float32),
        scratch_types=[
            pltpu.VMEM((BPW * S,), jnp.int32),
            pltpu.VMEM((BPW,), jnp.int32),
            pltpu.VMEM((BPW,), jnp.int32),
            pltpu.VMEM((BPW, D), jnp.float32),
            pltpu.VMEM((BPW, D), jnp.float32),
            pltpu.VMEM((D // 8, 8, BPW), jnp.float32),
            pltpu.VMEM((D // 8, 8, BPW), jnp.float32),
            pltpu.VMEM((S, D), jnp.float32),
            pltpu.SemaphoreType.DMA,
            pltpu.SemaphoreType.DMA,
        ],
        compiler_params=pltpu.CompilerParams(
            use_tc_tiling_on_sc=False, needs_layout_passes=False),
    )
    def gather_kernel(x_hbm, tbl_hbm, pos_hbm, out_hbm,
                      xb, ib0, ib1, gb0, gb1, ob0, ob1, pos_v, sem_g, sem_o):
        wid = lax.axis_index("s") * NC + lax.axis_index("c")
        b0 = wid * BPW
        ib = (ib0, ib1)
        gb = (gb0, gb1)
        ob = (ob0, ob1)
        iota = lax.iota(jnp.int32, L)

        pltpu.sync_copy(x_hbm.at[pl.ds(b0 * S, BPW * S)], xb)
        pltpu.sync_copy(pos_hbm, pos_v)

        def build_idx(s, buf):
            for j in range(BPW // L):
                g = plsc.load_gather(xb, [(iota + j * L) * S + s])
                buf[pl.ds(j * L, L)] = g

        def start_g(s, p):
            pltpu.async_copy(tbl_hbm.at[ib[p]], gb[p], sem_g)

        def wait_g(s, p):
            pltpu.make_async_copy(tbl_hbm.at[ib[p]], gb[p], sem_g).wait()

        def start_o(s, buf):
            pltpu.async_copy(buf, out_hbm.at[s, :, wid], sem_o)

        def wait_o(s, buf):
            pltpu.make_async_copy(buf, out_hbm.at[s, :, wid], sem_o).wait()

        def transpose_add(s, src, dst):
            def grp(jg, c):
                pv = pos_v[s, pl.ds(jg * L, L)]
                for l in range(L):
                    d = jg * L + l
                    bc = jnp.broadcast_to(pv[l], (L,))
                    dcol = jnp.broadcast_to(d, (L,))
                    for j in range(BPW // L):
                        g = plsc.load_gather(src, [iota + j * L, dcol])
                        dst[d >> 3, d & 7, pl.ds(j * L, L)] = g + bc
                return c
            lax.fori_loop(0, D // L, grp, 0)

        build_idx(0, ib[0])
        start_g(0, 0)
        build_idx(1, ib[1])

        def pair(t, carry):
            for k in range(2):
                s = 2 * t + k
                p = k
                lax.cond(s + 1 < S, lambda: start_g(s + 1, 1 - p),
                         lambda: None)
                wait_g(s, p)
                lax.cond(s >= 2, lambda: wait_o(s - 2, ob[p]), lambda: None)
                transpose_add(s, gb[p], ob[p])
                start_o(s, ob[p])
                lax.cond(s + 2 < S, lambda: build_idx(s + 2, ib[p]),
                         lambda: None)
            return carry

        lax.fori_loop(0, S // 2, pair, 0)
        wait_o(S - 2, ob[0])
        wait_o(S - 1, ob[1])

    return gather_kernel


def kernel(x, token_table, pos_embed):
    B, S = x.shape
    V, D = token_table.shape
    info = plsc.get_sparse_core_info()
    NC, NS = info.num_cores, info.num_subcores

    x_flat = x.reshape(B * S).astype(jnp.int32)
    pos2d = pos_embed[0, :S, :].astype(jnp.float32)
    tbl_t = token_table.T
    VT = (V // KC) * KC
    tail_lin = tbl_t[:, VT:].T.reshape((V - VT) * D)

    lin = _build_transpose(V, D, NC, NS)(tbl_t, tail_lin)
    out5 = _build_gather(B, S, D, V, NC, NS)(
        x_flat, lin.reshape(V, D), pos2d)
    return out5.transpose(2, 4, 0, 1, 3).reshape(B, S, D)


# speedup vs baseline: 1.1215x; 1.1215x over previous
import functools

import jax
import jax.numpy as jnp
from jax import lax
from jax.experimental import pallas as pl
from jax.experimental.pallas import tpu as pltpu
from jax.experimental.pallas import tpu_sc as plsc

L = 16
KC = 128


def _build_transpose(V, D, NC, NS):
    NW = NC * NS
    nblk = V // KC
    VT = nblk * KC
    tail = V - VT
    trips = (nblk + NW - 1) // NW
    if trips % 2:
        trips += 1
    ntr = D // 8

    mesh = plsc.VectorSubcoreMesh(core_axis_name="c", subcore_axis_name="s")

    @functools.partial(
        pl.kernel,
        mesh=mesh,
        out_type=jax.ShapeDtypeStruct((V * D,), jnp.float32),
        scratch_types=[
            pltpu.VMEM((D, KC), jnp.float32),
            pltpu.VMEM((D, KC), jnp.float32),
            pltpu.VMEM((KC, 65), jnp.float32),
            pltpu.VMEM((KC * D,), jnp.float32),
            pltpu.VMEM((KC * D,), jnp.float32),
            pltpu.VMEM((max(tail, 1) * D,), jnp.float32),
            pltpu.SemaphoreType.DMA,
            pltpu.SemaphoreType.DMA,
        ],
        compiler_params=pltpu.CompilerParams(needs_layout_passes=False),
    )
    def transpose_kernel(tt_hbm, tail_hbm, out_hbm,
                         tb0, tb1, mid, ob0, ob1, tlb, sem_i, sem_o):
        wid = lax.axis_index("s") * NC + lax.axis_index("c")
        tb = (tb0, tb1)
        ob = (ob0, ob1)
        iota = lax.iota(jnp.int32, L)
        pat65 = iota * 65

        def blk(i):
            return wid + NW * i

        def tiles(i, buf):
            c0 = blk(i) * KC
            pairs = []
            for tr in range(ntr):
                pairs.append((tt_hbm.at[pl.ds(tr * 8, 8), pl.ds(c0, KC)],
                              buf.at[pl.ds(tr * 8, 8), :]))
            return pairs

        def start_in(i, buf):
            @pl.when(blk(i) < nblk)
            def _():
                for s_, d_ in tiles(i, buf):
                    pltpu.async_copy(s_, d_, sem_i)

        def wait_in(i, buf):
            @pl.when(blk(i) < nblk)
            def _():
                for s_, d_ in tiles(i, buf):
                    pltpu.make_async_copy(s_, d_, sem_i).wait()

        def start_out(i, buf):
            @pl.when(blk(i) < nblk)
            def _():
                pltpu.async_copy(
                    buf, out_hbm.at[pl.ds(blk(i) * KC * D, KC * D)], sem_o)

        def wait_out(i, buf):
            @pl.when(blk(i) < nblk)
            def _():
                pltpu.make_async_copy(
                    buf, out_hbm.at[pl.ds(blk(i) * KC * D, KC * D)],
                    sem_o).wait()

        def transpose_block(src, dst):
            def hop_a(d, c):
                for c16 in range(KC // L):
                    v = src[d, pl.ds(c16 * L, L)]
                    plsc.store_scatter(mid, [iota + c16 * L,
                                             jnp.broadcast_to(d, (L,))], v)
                return c
            lax.fori_loop(0, D, hop_a, 0)

            def hop_b(v, c):
                for j in range(D // L):
                    dst[pl.ds(v * D + j * L, L)] = mid[v, pl.ds(j * L, L)]
                return c
            lax.fori_loop(0, KC, hop_b, 0)

        if tail:
            @pl.when(wid == NW - 1)
            def _():
                pltpu.sync_copy(tail_hbm, tlb)
                pltpu.sync_copy(tlb, out_hbm.at[pl.ds(VT * D, tail * D)])

        start_in(0, tb[0])

        def pair(t, carry):
            for k in range(2):
                i = 2 * t + k
                p = k
                start_in(i + 1, tb[1 - p])
                wait_in(i, tb[p])
                lax.cond(i >= 2, lambda: wait_out(i - 2, ob[p]),
                         lambda: None)
                transpose_block(tb[p], ob[p])
                start_out(i, ob[p])
            return carry

        lax.fori_loop(0, trips // 2, pair, 0)
        wait_out(trips - 2, ob[0])
        wait_out(trips - 1, ob[1])

    return transpose_kernel


def _build_gather(B, S, D, V, NC, NS):
    NW = NC * NS
    BPW = B // NW

    mesh = plsc.VectorSubcoreMesh(core_axis_name="c", subcore_axis_name="s")

    @functools.partial(
        pl.kernel,
        mesh=mesh,
        out_type=jax.ShapeDtypeStruct((S, D // 8, B // 128, 8, 128),
                                      jnp.float32),
        scratch_types=[
            pltpu.VMEM((BPW * S,), jnp.int32),
            pltpu.VMEM((BPW,), jnp.int32),
            pltpu.VMEM((BPW,), jnp.int32),
            pltpu.VMEM((BPW, D), jnp.float32),
            pltpu.VMEM((BPW, D), jnp.float32),
            pltpu.VMEM((D, BPW + 1), jnp.float32),
            pltpu.VMEM((D // 8, 8, BPW), jnp.float32),
            pltpu.VMEM((D // 8, 8, BPW), jnp.float32),
            pltpu.VMEM((S, D), jnp.float32),
            pltpu.SemaphoreType.DMA,
            pltpu.SemaphoreType.DMA,
        ],
        compiler_params=pltpu.CompilerParams(
            use_tc_tiling_on_sc=False, needs_layout_passes=False),
    )
    def gather_kernel(x_hbm, tbl_hbm, pos_hbm, out_hbm,
                      xb, ib0, ib1, gb0, gb1, mid, ob0, ob1, pos_v,
                      sem_g, sem_o):
        wid = lax.axis_index("s") * NC + lax.axis_index("c")
        b0 = wid * BPW
        ib = (ib0, ib1)
        gb = (gb0, gb1)
        ob = (ob0, ob1)
        iota = lax.iota(jnp.int32, L)

        pltpu.sync_copy(x_hbm.at[pl.ds(b0 * S, BPW * S)], xb)
        pltpu.sync_copy(pos_hbm, pos_v)

        def build_idx(s, buf):
            for j in range(BPW // L):
                g = plsc.load_gather(xb, [(iota + j * L) * S + s])
                buf[pl.ds(j * L, L)] = g

        def start_g(s, p):
            pltpu.async_copy(tbl_hbm.at[ib[p]], gb[p], sem_g)

        def wait_g(s, p):
            pltpu.make_async_copy(tbl_hbm.at[ib[p]], gb[p], sem_g).wait()

        def start_o(s, buf):
            pltpu.async_copy(buf, out_hbm.at[s, :, wid], sem_o)

        def wait_o(s, buf):
            pltpu.make_async_copy(buf, out_hbm.at[s, :, wid], sem_o).wait()

        def transpose_add(s, src, dst):
            pvs = [pos_v[s, pl.ds(jg * L, L)] for jg in range(D // L)]

            def hop_a(b, c):
                bb = jnp.broadcast_to(b, (L,))
                for jg in range(D // L):
                    v = src[b, pl.ds(jg * L, L)] + pvs[jg]
                    plsc.store_scatter(mid, [iota + jg * L, bb], v)
                return c
            lax.fori_loop(0, BPW, hop_a, 0)

            def hop_b(d, c):
                for j in range(BPW // L):
                    dst[d >> 3, d & 7, pl.ds(j * L, L)] = \
                        mid[d, pl.ds(j * L, L)]
                return c
            lax.fori_loop(0, D, hop_b, 0)

        build_idx(0, ib[0])
        start_g(0, 0)
        build_idx(1, ib[1])

        def pair(t, carry):
            for k in range(2):
                s = 2 * t + k
                p = k
                lax.cond(s + 1 < S, lambda: start_g(s + 1, 1 - p),
                         lambda: None)
                wait_g(s, p)
                lax.cond(s >= 2, lambda: wait_o(s - 2, ob[p]), lambda: None)
                transpose_add(s, gb[p], ob[p])
                start_o(s, ob[p])
                lax.cond(s + 2 < S, lambda: build_idx(s + 2, ib[p]),
                         lambda: None)
            return carry

        lax.fori_loop(0, S // 2, pair, 0)
        wait_o(S - 2, ob[0])
        wait_o(S - 1, ob[1])

    return gather_kernel


def kernel(x, token_table, pos_embed):
    B, S = x.shape
    V, D = token_table.shape
    info = plsc.get_sparse_core_info()
    NC, NS = info.num_cores, info.num_subcores

    x_flat = x.reshape(B * S).astype(jnp.int32)
    pos2d = pos_embed[0, :S, :].astype(jnp.float32)
    tbl_t = token_table.T
    VT = (V // KC) * KC
    tail_lin = tbl_t[:, VT:].T.reshape((V - VT) * D)

    lin = _build_transpose(V, D, NC, NS)(tbl_t, tail_lin)
    out5 = _build_gather(B, S, D, V, NC, NS)(
        x_flat, lin.reshape(V, D), pos2d)
    return out5.transpose(2, 4, 0, 1, 3).reshape(B, S, D)


# speedup vs baseline: 2.4364x; 2.1724x over previous
import functools

import jax
import jax.numpy as jnp
from jax import lax
from jax.experimental import pallas as pl
from jax.experimental.pallas import tpu as pltpu
from jax.experimental.pallas import tpu_sc as plsc

L = 16
KC = 128


def _build_transpose(V, D, NC, NS):
    NW = NC * NS
    nblk = V // KC
    VT = nblk * KC
    tail = V - VT
    trips = (nblk + NW - 1) // NW
    if trips % 2:
        trips += 1
    ntr = D // 8

    mesh = plsc.VectorSubcoreMesh(core_axis_name="c", subcore_axis_name="s")

    @functools.partial(
        pl.kernel,
        mesh=mesh,
        out_type=jax.ShapeDtypeStruct((V * D,), jnp.float32),
        scratch_types=[
            pltpu.VMEM((D, KC), jnp.float32),
            pltpu.VMEM((D, KC), jnp.float32),
            pltpu.VMEM((KC, 65), jnp.float32),
            pltpu.VMEM((KC * D,), jnp.float32),
            pltpu.VMEM((KC * D,), jnp.float32),
            pltpu.VMEM((max(tail, 1) * D,), jnp.float32),
            pltpu.SemaphoreType.DMA,
            pltpu.SemaphoreType.DMA,
        ],
        compiler_params=pltpu.CompilerParams(needs_layout_passes=False),
    )
    def transpose_kernel(tt_hbm, tail_hbm, out_hbm,
                         tb0, tb1, mid, ob0, ob1, tlb, sem_i, sem_o):
        wid = lax.axis_index("s") * NC + lax.axis_index("c")
        tb = (tb0, tb1)
        ob = (ob0, ob1)
        iota = lax.iota(jnp.int32, L)
        pat65 = iota * 65

        def blk(i):
            return wid + NW * i

        def tiles(i, buf):
            c0 = blk(i) * KC
            pairs = []
            for tr in range(ntr):
                pairs.append((tt_hbm.at[pl.ds(tr * 8, 8), pl.ds(c0, KC)],
                              buf.at[pl.ds(tr * 8, 8), :]))
            return pairs

        def start_in(i, buf):
            @pl.when(blk(i) < nblk)
            def _():
                for s_, d_ in tiles(i, buf):
                    pltpu.async_copy(s_, d_, sem_i)

        def wait_in(i, buf):
            @pl.when(blk(i) < nblk)
            def _():
                for s_, d_ in tiles(i, buf):
                    pltpu.make_async_copy(s_, d_, sem_i).wait()

        def start_out(i, buf):
            @pl.when(blk(i) < nblk)
            def _():
                pltpu.async_copy(
                    buf, out_hbm.at[pl.ds(blk(i) * KC * D, KC * D)], sem_o)

        def wait_out(i, buf):
            @pl.when(blk(i) < nblk)
            def _():
                pltpu.make_async_copy(
                    buf, out_hbm.at[pl.ds(blk(i) * KC * D, KC * D)],
                    sem_o).wait()

        def transpose_block(src, dst):
            @plsc.parallel_loop(0, D, 1, unroll=8)
            def hop_a(d):
                db = jnp.broadcast_to(d, (L,))
                for c16 in range(KC // L):
                    v = src[d, pl.ds(c16 * L, L)]
                    plsc.store_scatter(mid, [iota + c16 * L, db], v)

            @plsc.parallel_loop(0, KC, 1, unroll=8)
            def hop_b(v):
                for j in range(D // L):
                    dst[pl.ds(v * D + j * L, L)] = mid[v, pl.ds(j * L, L)]

        if tail:
            @pl.when(wid == NW - 1)
            def _():
                pltpu.sync_copy(tail_hbm, tlb)
                pltpu.sync_copy(tlb, out_hbm.at[pl.ds(VT * D, tail * D)])

        start_in(0, tb[0])

        def pair(t, carry):
            for k in range(2):
                i = 2 * t + k
                p = k
                start_in(i + 1, tb[1 - p])
                wait_in(i, tb[p])
                lax.cond(i >= 2, lambda: wait_out(i - 2, ob[p]),
                         lambda: None)
                transpose_block(tb[p], ob[p])
                start_out(i, ob[p])
            return carry

        lax.fori_loop(0, trips // 2, pair, 0)
        wait_out(trips - 2, ob[0])
        wait_out(trips - 1, ob[1])

    return transpose_kernel


def _build_gather(B, S, D, V, NC, NS):
    NW = NC * NS
    BPW = B // NW

    mesh = plsc.VectorSubcoreMesh(core_axis_name="c", subcore_axis_name="s")

    @functools.partial(
        pl.kernel,
        mesh=mesh,
        out_type=jax.ShapeDtypeStruct((S, D // 8, B // 128, 8, 128),
                                      jnp.float32),
        scratch_types=[
            pltpu.VMEM((BPW * S,), jnp.int32),
            pltpu.VMEM((BPW,), jnp.int32),
            pltpu.VMEM((BPW,), jnp.int32),
            pltpu.VMEM((BPW, D), jnp.float32),
            pltpu.VMEM((BPW, D), jnp.float32),
            pltpu.VMEM((D, BPW + 1), jnp.float32),
            pltpu.VMEM((D // 8, 8, BPW), jnp.float32),
            pltpu.VMEM((D // 8, 8, BPW), jnp.float32),
            pltpu.VMEM((S, D), jnp.float32),
            pltpu.SemaphoreType.DMA,
            pltpu.SemaphoreType.DMA,
        ],
        compiler_params=pltpu.CompilerParams(
            use_tc_tiling_on_sc=False, needs_layout_passes=False),
    )
    def gather_kernel(x_hbm, tbl_hbm, pos_hbm, out_hbm,
                      xb, ib0, ib1, gb0, gb1, mid, ob0, ob1, pos_v,
                      sem_g, sem_o):
        wid = lax.axis_index("s") * NC + lax.axis_index("c")
        b0 = wid * BPW
        ib = (ib0, ib1)
        gb = (gb0, gb1)
        ob = (ob0, ob1)
        iota = lax.iota(jnp.int32, L)

        pltpu.sync_copy(x_hbm.at[pl.ds(b0 * S, BPW * S)], xb)
        pltpu.sync_copy(pos_hbm, pos_v)

        def build_idx(s, buf):
            for j in range(BPW // L):
                g = plsc.load_gather(xb, [(iota + j * L) * S + s])
                buf[pl.ds(j * L, L)] = g

        def start_g(s, p):
            pltpu.async_copy(tbl_hbm.at[ib[p]], gb[p], sem_g)

        def wait_g(s, p):
            pltpu.make_async_copy(tbl_hbm.at[ib[p]], gb[p], sem_g).wait()

        def start_o(s, buf):
            pltpu.async_copy(buf, out_hbm.at[s, :, wid], sem_o)

        def wait_o(s, buf):
            pltpu.make_async_copy(buf, out_hbm.at[s, :, wid], sem_o).wait()

        def transpose_add(s, src, dst):
            pvs = [pos_v[s, pl.ds(jg * L, L)] for jg in range(D // L)]

            @plsc.parallel_loop(0, BPW, 1, unroll=8)
            def hop_a(b):
                bb = jnp.broadcast_to(b, (L,))
                for jg in range(D // L):
                    v = src[b, pl.ds(jg * L, L)] + pvs[jg]
                    plsc.store_scatter(mid, [iota + jg * L, bb], v)

            @plsc.parallel_loop(0, D, 1, unroll=8)
            def hop_b(d):
                for j in range(BPW // L):
                    dst[d >> 3, d & 7, pl.ds(j * L, L)] = \
                        mid[d, pl.ds(j * L, L)]

        build_idx(0, ib[0])
        start_g(0, 0)
        build_idx(1, ib[1])

        def pair(t, carry):
            for k in range(2):
                s = 2 * t + k
                p = k
                lax.cond(s + 1 < S, lambda: start_g(s + 1, 1 - p),
                         lambda: None)
                wait_g(s, p)
                lax.cond(s >= 2, lambda: wait_o(s - 2, ob[p]), lambda: None)
                transpose_add(s, gb[p], ob[p])
                start_o(s, ob[p])
                lax.cond(s + 2 < S, lambda: build_idx(s + 2, ib[p]),
                         lambda: None)
            return carry

        lax.fori_loop(0, S // 2, pair, 0)
        wait_o(S - 2, ob[0])
        wait_o(S - 1, ob[1])

    return gather_kernel


def kernel(x, token_table, pos_embed):
    B, S = x.shape
    V, D = token_table.shape
    info = plsc.get_sparse_core_info()
    NC, NS = info.num_cores, info.num_subcores

    x_flat = x.reshape(B * S).astype(jnp.int32)
    pos2d = pos_embed[0, :S, :].astype(jnp.float32)
    tbl_t = token_table.T
    VT = (V // KC) * KC
    tail_lin = tbl_t[:, VT:].T.reshape((V - VT) * D)

    lin = _build_transpose(V, D, NC, NS)(tbl_t, tail_lin)
    out5 = _build_gather(B, S, D, V, NC, NS)(
        x_flat, lin.reshape(V, D), pos2d)
    return out5.transpose(2, 4, 0, 1, 3).reshape(B, S, D)
